# baseline (speedup 1.0000x reference)
import functools

import jax
import jax.numpy as jnp
from jax import lax
from jax.experimental import pallas as pl
from jax.experimental.pallas import tpu as pltpu
from jax.experimental.pallas import tpu_sc as plsc

N = 10000
E = 320000
HID = 128
CAT = 256
NPAD = 10240
NC = 2
NS = 16
K = 128
NB = 2
CPS = 160
SLAB = 40
EPAD = NS * CPS * K
NCHUNK = EPAD // K
RPW = NPAD // NS
BR = 512

_mesh = lambda: plsc.VectorSubcoreMesh(
    core_axis_name="c", subcore_axis_name="s",
    num_cores=NC, num_subcores=NS)


def _edge_pipe(table_hbm, src_slab, dst_slab, acc, rows, sems, nchunks):
    pltpu.async_copy(table_hbm.at[src_slab.at[0]], rows[0], sems[0])

    def group(g, carry):
        for b in range(NB):
            i = g * NB + b
            pltpu.make_async_copy(table_hbm.at[src_slab.at[i]], rows[b],
                                  sems[b]).wait()
            nb = (b + 1) % NB

            @pl.when(i + 1 < nchunks)
            def _():
                pltpu.async_copy(table_hbm.at[src_slab.at[i + 1]], rows[nb],
                                 sems[nb])

            pltpu.sync_copy(rows[b], acc.at[dst_slab.at[i]], add=True)
        return carry

    lax.fori_loop(0, nchunks // NB, group, 0)


def _sc_deg(dst2, zdeg):
    cpw = NCHUNK // (NC * NS)

    @functools.partial(
        pl.kernel,
        out_type=jax.ShapeDtypeStruct((NC * NS, NPAD), jnp.float32),
        mesh=_mesh(),
        scratch_types=[
            pltpu.VMEM((SLAB, K), jnp.int32),
            pltpu.VMEM((NPAD,), jnp.float32),
        ],
        compiler_params=pltpu.CompilerParams(needs_layout_passes=False),
    )
    def run(dst_hbm, zdeg_hbm, out_deg, dst_slab, degp):
        c = lax.axis_index("c")
        s = lax.axis_index("s")
        w = c * NS + s
        ones16 = jnp.ones((16,), jnp.float32)
        pltpu.sync_copy(zdeg_hbm, degp)
        for h in range(cpw // SLAB):
            pltpu.sync_copy(dst_hbm.at[pl.ds(w * cpw + h * SLAB, SLAB)],
                            dst_slab)

            def body(i, carry):
                for j in range(K // 16):
                    idx16 = dst_slab[i, pl.ds(j * 16, 16)]
                    plsc.addupdate_scatter(degp, [idx16], ones16)
                return carry

            lax.fori_loop(0, SLAB, body, 0)
        pltpu.sync_copy(degp, out_deg.at[w])

    return run(dst2, zdeg)


def _sc_gate(t0, t1, src2, dst2, zrow):
    @functools.partial(
        pl.kernel,
        out_type=(jax.ShapeDtypeStruct((NPAD, HID), jnp.float32),
                  jax.ShapeDtypeStruct((NPAD, HID), jnp.float32)),
        mesh=_mesh(),
        scratch_types=[
            pltpu.VMEM((SLAB, K), jnp.int32),
            pltpu.VMEM((SLAB, K), jnp.int32),
            [pltpu.VMEM((K, HID), jnp.float32)] * NB,
            pltpu.VMEM_SHARED((NPAD, HID), jnp.float32),
            [pltpu.SemaphoreType.DMA] * NB,
        ],
        compiler_params=pltpu.CompilerParams(needs_layout_passes=False),
    )
    def run(t0_hbm, t1_hbm, src_hbm, dst_hbm, zrow_hbm,
            out0, out1, src_slab, dst_slab, rows, acc, sems):
        c = lax.axis_index("c")
        s = lax.axis_index("s")
        r0 = s * RPW
        pltpu.sync_copy(zrow_hbm, acc.at[pl.ds(r0, RPW)])
        plsc.subcore_barrier()

        for h in range(CPS // SLAB):
            base = s * CPS + h * SLAB
            pltpu.sync_copy(src_hbm.at[pl.ds(base, SLAB)], src_slab)
            pltpu.sync_copy(dst_hbm.at[pl.ds(base, SLAB)], dst_slab)

            @pl.when(c == 0)
            def _():
                _edge_pipe(t0_hbm, src_slab, dst_slab, acc, rows, sems, SLAB)

            @pl.when(c == 1)
            def _():
                _edge_pipe(t1_hbm, src_slab, dst_slab, acc, rows, sems, SLAB)

        plsc.subcore_barrier()

        @pl.when(c == 0)
        def _():
            pltpu.sync_copy(acc.at[pl.ds(r0, RPW)], out0.at[pl.ds(r0, RPW)])

        @pl.when(c == 1)
        def _():
            pltpu.sync_copy(acc.at[pl.ds(r0, RPW)], out1.at[pl.ds(r0, RPW)])

    return run(t0, t1, src2, dst2, zrow)


def _sc_cand(t, src2, dst2, zrow):
    cpw = NCHUNK // (NC * NS)

    @functools.partial(
        pl.kernel,
        out_type=(jax.ShapeDtypeStruct((NPAD, HID), jnp.float32),
                  jax.ShapeDtypeStruct((NPAD, HID), jnp.float32)),
        mesh=_mesh(),
        scratch_types=[
            pltpu.VMEM((SLAB, K), jnp.int32),
            pltpu.VMEM((SLAB, K), jnp.int32),
            [pltpu.VMEM((K, HID), jnp.float32)] * NB,
            pltpu.VMEM_SHARED((NPAD, HID), jnp.float32),
            [pltpu.SemaphoreType.DMA] * NB,
        ],
        compiler_params=pltpu.CompilerParams(needs_layout_passes=False),
    )
    def run(t_hbm, src_hbm, dst_hbm, zrow_hbm,
            out0, out1, src_slab, dst_slab, rows, acc, sems):
        c = lax.axis_index("c")
        s = lax.axis_index("s")
        r0 = s * RPW
        w = c * NS + s
        pltpu.sync_copy(zrow_hbm, acc.at[pl.ds(r0, RPW)])
        plsc.subcore_barrier()
        for h in range(cpw // SLAB):
            base = w * cpw + h * SLAB
            pltpu.sync_copy(src_hbm.at[pl.ds(base, SLAB)], src_slab)
            pltpu.sync_copy(dst_hbm.at[pl.ds(base, SLAB)], dst_slab)
            _edge_pipe(t_hbm, src_slab, dst_slab, acc, rows, sems, SLAB)
        plsc.subcore_barrier()

        @pl.when(c == 0)
        def _():
            pltpu.sync_copy(acc.at[pl.ds(r0, RPW)], out0.at[pl.ds(r0, RPW)])

        @pl.when(c == 1)
        def _():
            pltpu.sync_copy(acc.at[pl.ds(r0, RPW)], out1.at[pl.ds(r0, RPW)])

    return run(t, src2, dst2, zrow)


def _dot(a, b):
    return jnp.dot(a, b, preferred_element_type=jnp.float32)


def _stage_a(x, st, wng, wsg, wnc_t, wsc_t, bg):
    def body(x_ref, s_ref, wng_ref, wsg_ref, wnct_ref, wsct_ref, bg_ref,
             t0_ref, t1_ref, sg_ref, p_ref, sc0_ref):
        xb = x_ref[...]
        sb = s_ref[...]
        wng_b = wng_ref[...]
        wsg_b = wsg_ref[...]
        yg = _dot(xb, wng_b[:HID]) + _dot(sb, wng_b[HID:])
        t0_ref[...] = yg[:, :HID]
        t1_ref[...] = yg[:, HID:]
        sg_ref[...] = _dot(xb, wsg_b[:HID]) + _dot(sb, wsg_b[HID:]) + bg_ref[...]
        p_ref[...] = _dot(xb, wnct_ref[...])
        sc0_ref[...] = _dot(xb, wsct_ref[...])

    full = lambda shape: pl.BlockSpec(shape, lambda i: (0, 0))
    rows = lambda w: pl.BlockSpec((BR, w), lambda i: (i, 0))
    return pl.pallas_call(
        body,
        grid=(NPAD // BR,),
        in_specs=[rows(HID), rows(HID), full((CAT, CAT)), full((CAT, CAT)),
                  full((HID, HID)), full((HID, HID)), full((1, CAT))],
        out_specs=[rows(HID), rows(HID), rows(CAT), rows(HID), rows(HID)],
        out_shape=[jax.ShapeDtypeStruct((NPAD, HID), jnp.float32),
                   jax.ShapeDtypeStruct((NPAD, HID), jnp.float32),
                   jax.ShapeDtypeStruct((NPAD, CAT), jnp.float32),
                   jax.ShapeDtypeStruct((NPAD, HID), jnp.float32),
                   jax.ShapeDtypeStruct((NPAD, HID), jnp.float32)],
    )(x, st, wng, wsg, wnc_t, wsc_t, bg)


def _stage_b(sg, g0, g1, dp, st, p, sc0, wnc_b, wsc_b, bc):
    def body(sg_ref, g0_ref, g1_ref, dp_ref, s_ref, p_ref, sc0_ref,
             wncb_ref, wscb_ref, bc_ref, t_ref, sc_ref, u_ref, inv_ref):
        deg = lax.dot_general(dp_ref[...], jnp.ones((NC * NS, 1), jnp.float32),
                              (((0,), (0,)), ((), ())),
                              preferred_element_type=jnp.float32)
        inv = 1.0 / jnp.maximum(deg, 1.0)
        agg = jnp.concatenate([g0_ref[...], g1_ref[...]], axis=1)
        h = jax.nn.sigmoid(sg_ref[...] + agg * inv)
        r = h[:, :HID]
        u = h[:, HID:]
        rs = r * s_ref[...]
        t_ref[...] = p_ref[...] + _dot(rs, wncb_ref[...])
        sc_ref[...] = sc0_ref[...] + _dot(rs, wscb_ref[...]) + bc_ref[...]
        u_ref[...] = u
        inv_ref[...] = jnp.broadcast_to(inv, (BR, HID))

    full = lambda shape: pl.BlockSpec(shape, lambda i: (0, 0))
    rows = lambda w: pl.BlockSpec((BR, w), lambda i: (i, 0))
    return pl.pallas_call(
        body,
        grid=(NPAD // BR,),
        in_specs=[rows(CAT), rows(HID), rows(HID),
                  pl.BlockSpec((NC * NS, BR), lambda i: (0, i)), rows(HID),
                  rows(HID), rows(HID), full((HID, HID)), full((HID, HID)),
                  full((1, HID))],
        out_specs=[rows(HID), rows(HID), rows(HID), rows(HID)],
        out_shape=[jax.ShapeDtypeStruct((NPAD, HID), jnp.float32),
                   jax.ShapeDtypeStruct((NPAD, HID), jnp.float32),
                   jax.ShapeDtypeStruct((NPAD, HID), jnp.float32),
                   jax.ShapeDtypeStruct((NPAD, HID), jnp.float32)],
    )(sg, g0, g1, dp, st, p, sc0, wnc_b, wsc_b, bc)


def _stage_c(a0, a1, sc, u, st, invb):
    def body(a0_ref, a1_ref, sc_ref, u_ref, s_ref, inv_ref, out_ref):
        agg = a0_ref[...] + a1_ref[...]
        cc = jnp.tanh(sc_ref[...] + agg * inv_ref[...])
        ub = u_ref[...]
        out_ref[...] = ub * s_ref[...] + (1.0 - ub) * cc

    rows = lambda w: pl.BlockSpec((BR, w), lambda i: (i, 0))
    return pl.pallas_call(
        body,
        grid=(NPAD // BR,),
        in_specs=[rows(HID)] * 6,
        out_specs=rows(HID),
        out_shape=jax.ShapeDtypeStruct((NPAD, HID), jnp.float32),
    )(a0, a1, sc, u, st, invb)


def kernel(edge_index, inputs, state, W_self_gate, W_neigh_gate, b_gate,
           gate_bias, W_self_cand, W_neigh_cand, b_cand, candidate_bias):
    src = edge_index[0].astype(jnp.int32)
    dst = edge_index[1].astype(jnp.int32)
    src_p = jnp.concatenate([src, jnp.zeros((EPAD - E,), jnp.int32)])
    dst_p = jnp.concatenate([dst, jnp.full((EPAD - E,), N, jnp.int32)])
    src2 = src_p.reshape(NCHUNK, K)
    dst2 = dst_p.reshape(NCHUNK, K)
    x = jnp.pad(inputs, ((0, NPAD - N), (0, 0)))
    st = jnp.pad(state, ((0, NPAD - N), (0, 0)))
    bg = (b_gate + gate_bias).reshape(1, CAT)
    bc = (b_cand + candidate_bias).reshape(1, HID)
    zrow = jnp.zeros((RPW, HID), jnp.float32)
    zdeg = jnp.zeros((NPAD,), jnp.float32)

    dp = _sc_deg(dst2, zdeg)
    t0, t1, sg, p, sc0 = _stage_a(
        x, st, W_neigh_gate, W_self_gate,
        W_neigh_cand[:HID], W_self_cand[:HID], bg)
    g0, g1 = _sc_gate(t0, t1, src2, dst2, zrow)
    tc, sc, u, invb = _stage_b(
        sg, g0, g1, dp, st, p, sc0, W_neigh_cand[HID:], W_self_cand[HID:], bc)
    a0, a1 = _sc_cand(tc, src2, dst2, zrow)
    new = _stage_c(a0, a1, sc, u, st, invb)
    return new[:N]


# speedup vs baseline: 3.0116x; 3.0116x over previous
import functools

import jax
import jax.numpy as jnp
from jax import lax
from jax.experimental import pallas as pl
from jax.experimental.pallas import tpu as pltpu
from jax.experimental.pallas import tpu_sc as plsc

N = 10000
E = 320000
HID = 128
CAT = 256
NPAD = 10240
NC = 2
NS = 16
K = 64
NB = 4
CPS = 320
SLAB = 40
EPAD = NS * CPS * K
NCHUNK = EPAD // K
RPW = NPAD // NS
BR = 512

_mesh = lambda: plsc.VectorSubcoreMesh(
    core_axis_name="c", subcore_axis_name="s",
    num_cores=NC, num_subcores=NS)


def _edge_pipe(table_hbm, src_slab, dst_slab, acc, rows, sems, nchunks,
               count_deg=None):
    for b in range(NB - 1):
        pltpu.async_copy(table_hbm.at[src_slab.at[b]], rows[b], sems[b])

    def group(g, carry):
        for b in range(NB):
            i = g * NB + b
            pltpu.make_async_copy(table_hbm.at[src_slab.at[i]], rows[b],
                                  sems[b]).wait()
            nxt = i + NB - 1
            nb = (b + NB - 1) % NB

            @pl.when(nxt < nchunks)
            def _():
                pltpu.async_copy(table_hbm.at[src_slab.at[nxt]], rows[nb],
                                 sems[nb])

            pltpu.sync_copy(rows[b], acc.at[dst_slab.at[i]], add=True)
            if count_deg is not None:
                ones16 = jnp.ones((16,), jnp.float32)
                for j in range(K // 16):
                    idx16 = dst_slab[i, pl.ds(j * 16, 16)]
                    plsc.addupdate_scatter(count_deg, [idx16], ones16)
        return carry

    lax.fori_loop(0, nchunks // NB, group, 0)


def _sc_deg(dst2, zdeg):
    cpw = NCHUNK // (NC * NS)

    @functools.partial(
        pl.kernel,
        out_type=jax.ShapeDtypeStruct((NC * NS, NPAD), jnp.float32),
        mesh=_mesh(),
        scratch_types=[
            pltpu.VMEM((SLAB, K), jnp.int32),
            pltpu.VMEM((NPAD,), jnp.float32),
        ],
        compiler_params=pltpu.CompilerParams(needs_layout_passes=False),
    )
    def run(dst_hbm, zdeg_hbm, out_deg, dst_slab, degp):
        c = lax.axis_index("c")
        s = lax.axis_index("s")
        w = c * NS + s
        ones16 = jnp.ones((16,), jnp.float32)
        pltpu.sync_copy(zdeg_hbm, degp)
        for h in range(cpw // SLAB):
            pltpu.sync_copy(dst_hbm.at[pl.ds(w * cpw + h * SLAB, SLAB)],
                            dst_slab)

            def body(i, carry):
                for j in range(K // 16):
                    idx16 = dst_slab[i, pl.ds(j * 16, 16)]
                    plsc.addupdate_scatter(degp, [idx16], ones16)
                return carry

            lax.fori_loop(0, SLAB, body, 0)
        pltpu.sync_copy(degp, out_deg.at[w])

    return run(dst2, zdeg)


def _sc_gate(t0, t1, src2, dst2, zrow):
    @functools.partial(
        pl.kernel,
        out_type=(jax.ShapeDtypeStruct((NPAD, HID), jnp.float32),
                  jax.ShapeDtypeStruct((NPAD, HID), jnp.float32)),
        mesh=_mesh(),
        scratch_types=[
            pltpu.VMEM((SLAB, K), jnp.int32),
            pltpu.VMEM((SLAB, K), jnp.int32),
            [pltpu.VMEM((K, HID), jnp.float32)] * NB,
            pltpu.VMEM_SHARED((NPAD, HID), jnp.float32),
            [pltpu.SemaphoreType.DMA] * NB,
        ],
        compiler_params=pltpu.CompilerParams(needs_layout_passes=False),
    )
    def run(t0_hbm, t1_hbm, src_hbm, dst_hbm, zrow_hbm,
            out0, out1, src_slab, dst_slab, rows, acc, sems):
        c = lax.axis_index("c")
        s = lax.axis_index("s")
        r0 = s * RPW
        pltpu.sync_copy(zrow_hbm, acc.at[pl.ds(r0, RPW)])
        plsc.subcore_barrier()

        for h in range(CPS // SLAB):
            base = s * CPS + h * SLAB
            pltpu.sync_copy(src_hbm.at[pl.ds(base, SLAB)], src_slab)
            pltpu.sync_copy(dst_hbm.at[pl.ds(base, SLAB)], dst_slab)

            @pl.when(c == 0)
            def _():
                _edge_pipe(t0_hbm, src_slab, dst_slab, acc, rows, sems, SLAB)

            @pl.when(c == 1)
            def _():
                _edge_pipe(t1_hbm, src_slab, dst_slab, acc, rows, sems, SLAB)

        plsc.subcore_barrier()

        @pl.when(c == 0)
        def _():
            pltpu.sync_copy(acc.at[pl.ds(r0, RPW)], out0.at[pl.ds(r0, RPW)])

        @pl.when(c == 1)
        def _():
            pltpu.sync_copy(acc.at[pl.ds(r0, RPW)], out1.at[pl.ds(r0, RPW)])

    return run(t0, t1, src2, dst2, zrow)


def _sc_cand(t, src2, dst2, zrow):
    cpw = NCHUNK // (NC * NS)

    @functools.partial(
        pl.kernel,
        out_type=(jax.ShapeDtypeStruct((NPAD, HID), jnp.float32),
                  jax.ShapeDtypeStruct((NPAD, HID), jnp.float32)),
        mesh=_mesh(),
        scratch_types=[
            pltpu.VMEM((SLAB, K), jnp.int32),
            pltpu.VMEM((SLAB, K), jnp.int32),
            [pltpu.VMEM((K, HID), jnp.float32)] * NB,
            pltpu.VMEM_SHARED((NPAD, HID), jnp.float32),
            [pltpu.SemaphoreType.DMA] * NB,
        ],
        compiler_params=pltpu.CompilerParams(needs_layout_passes=False),
    )
    def run(t_hbm, src_hbm, dst_hbm, zrow_hbm,
            out0, out1, src_slab, dst_slab, rows, acc, sems):
        c = lax.axis_index("c")
        s = lax.axis_index("s")
        r0 = s * RPW
        w = c * NS + s
        pltpu.sync_copy(zrow_hbm, acc.at[pl.ds(r0, RPW)])
        plsc.subcore_barrier()
        for h in range(cpw // SLAB):
            base = w * cpw + h * SLAB
            pltpu.sync_copy(src_hbm.at[pl.ds(base, SLAB)], src_slab)
            pltpu.sync_copy(dst_hbm.at[pl.ds(base, SLAB)], dst_slab)
            _edge_pipe(t_hbm, src_slab, dst_slab, acc, rows, sems, SLAB)
        plsc.subcore_barrier()

        @pl.when(c == 0)
        def _():
            pltpu.sync_copy(acc.at[pl.ds(r0, RPW)], out0.at[pl.ds(r0, RPW)])

        @pl.when(c == 1)
        def _():
            pltpu.sync_copy(acc.at[pl.ds(r0, RPW)], out1.at[pl.ds(r0, RPW)])

    return run(t, src2, dst2, zrow)


def _dot(a, b):
    return jnp.dot(a, b, preferred_element_type=jnp.float32)


def _stage_a(x, st, wng, wsg, wnc_t, wsc_t, bg):
    def body(x_ref, s_ref, wng_ref, wsg_ref, wnct_ref, wsct_ref, bg_ref,
             t0_ref, t1_ref, sg_ref, p_ref, sc0_ref):
        xb = x_ref[...]
        sb = s_ref[...]
        wng_b = wng_ref[...]
        wsg_b = wsg_ref[...]
        yg = _dot(xb, wng_b[:HID]) + _dot(sb, wng_b[HID:])
        t0_ref[...] = yg[:, :HID]
        t1_ref[...] = yg[:, HID:]
        sg_ref[...] = _dot(xb, wsg_b[:HID]) + _dot(sb, wsg_b[HID:]) + bg_ref[...]
        p_ref[...] = _dot(xb, wnct_ref[...])
        sc0_ref[...] = _dot(xb, wsct_ref[...])

    full = lambda shape: pl.BlockSpec(shape, lambda i: (0, 0))
    rows = lambda w: pl.BlockSpec((BR, w), lambda i: (i, 0))
    return pl.pallas_call(
        body,
        grid=(NPAD // BR,),
        in_specs=[rows(HID), rows(HID), full((CAT, CAT)), full((CAT, CAT)),
                  full((HID, HID)), full((HID, HID)), full((1, CAT))],
        out_specs=[rows(HID), rows(HID), rows(CAT), rows(HID), rows(HID)],
        out_shape=[jax.ShapeDtypeStruct((NPAD, HID), jnp.float32),
                   jax.ShapeDtypeStruct((NPAD, HID), jnp.float32),
                   jax.ShapeDtypeStruct((NPAD, CAT), jnp.float32),
                   jax.ShapeDtypeStruct((NPAD, HID), jnp.float32),
                   jax.ShapeDtypeStruct((NPAD, HID), jnp.float32)],
    )(x, st, wng, wsg, wnc_t, wsc_t, bg)


def _stage_b(sg, g0, g1, dp, st, p, sc0, wnc_b, wsc_b, bc):
    def body(sg_ref, g0_ref, g1_ref, dp_ref, s_ref, p_ref, sc0_ref,
             wncb_ref, wscb_ref, bc_ref, t_ref, sc_ref, u_ref):
        deg = lax.dot_general(dp_ref[...],
                              jnp.ones((NC * NS, 1), jnp.float32),
                              (((0,), (0,)), ((), ())),
                              preferred_element_type=jnp.float32)
        inv = 1.0 / jnp.maximum(deg, 1.0)
        agg = jnp.concatenate([g0_ref[...], g1_ref[...]], axis=1)
        h = jax.nn.sigmoid(sg_ref[...] + agg * inv)
        r = h[:, :HID]
        u = h[:, HID:]
        rs = r * s_ref[...]
        t_ref[...] = p_ref[...] + _dot(rs, wncb_ref[...])
        sc_ref[...] = sc0_ref[...] + _dot(rs, wscb_ref[...]) + bc_ref[...]
        u_ref[...] = u

    full = lambda shape: pl.BlockSpec(shape, lambda i: (0, 0))
    rows = lambda w: pl.BlockSpec((BR, w), lambda i: (i, 0))
    return pl.pallas_call(
        body,
        grid=(NPAD // BR,),
        in_specs=[rows(CAT), rows(HID), rows(HID),
                  pl.BlockSpec((NC * NS, BR), lambda i: (0, i)), rows(HID),
                  rows(HID), rows(HID), full((HID, HID)), full((HID, HID)),
                  full((1, HID))],
        out_specs=[rows(HID), rows(HID), rows(HID)],
        out_shape=[jax.ShapeDtypeStruct((NPAD, HID), jnp.float32),
                   jax.ShapeDtypeStruct((NPAD, HID), jnp.float32),
                   jax.ShapeDtypeStruct((NPAD, HID), jnp.float32)],
    )(sg, g0, g1, dp, st, p, sc0, wnc_b, wsc_b, bc)


def _stage_c(a0, a1, sc, u, st, dp):
    def body(a0_ref, a1_ref, sc_ref, u_ref, s_ref, dp_ref, out_ref):
        deg = lax.dot_general(dp_ref[...],
                              jnp.ones((NC * NS, 1), jnp.float32),
                              (((0,), (0,)), ((), ())),
                              preferred_element_type=jnp.float32)
        inv = 1.0 / jnp.maximum(deg, 1.0)
        agg = a0_ref[...] + a1_ref[...]
        cc = jnp.tanh(sc_ref[...] + agg * inv)
        ub = u_ref[...]
        out_ref[...] = ub * s_ref[...] + (1.0 - ub) * cc

    rows = lambda w: pl.BlockSpec((BR, w), lambda i: (i, 0))
    return pl.pallas_call(
        body,
        grid=(NPAD // BR,),
        in_specs=[rows(HID)] * 5
        + [pl.BlockSpec((NC * NS, BR), lambda i: (0, i))],
        out_specs=rows(HID),
        out_shape=jax.ShapeDtypeStruct((NPAD, HID), jnp.float32),
    )(a0, a1, sc, u, st, dp)


def kernel(edge_index, inputs, state, W_self_gate, W_neigh_gate, b_gate,
           gate_bias, W_self_cand, W_neigh_cand, b_cand, candidate_bias):
    src = edge_index[0].astype(jnp.int32)
    dst = edge_index[1].astype(jnp.int32)
    pad_i = jnp.arange(EPAD - E, dtype=jnp.int32)
    src_p = jnp.concatenate([src, pad_i % N])
    dst_p = jnp.concatenate([dst, N + pad_i % (NPAD - N)])
    src2 = src_p.reshape(NCHUNK, K)
    dst2 = dst_p.reshape(NCHUNK, K)
    st = jnp.pad(state, ((0, NPAD - N), (0, 0)))
    bg = (b_gate + gate_bias).reshape(1, CAT)
    bc = (b_cand + candidate_bias).reshape(1, HID)
    zrow = jnp.zeros((RPW, HID), jnp.float32)
    zdeg = jnp.zeros((NPAD,), jnp.float32)

    dp = _sc_deg(dst2, zdeg)
    t0, t1, sg, p, sc0 = _stage_a(
        inputs, st, W_neigh_gate, W_self_gate,
        W_neigh_cand[:HID], W_self_cand[:HID], bg)
    g0, g1 = _sc_gate(t0, t1, src2, dst2, zrow)
    tc, sc, u = _stage_b(
        sg, g0, g1, dp, st, p, sc0,
        W_neigh_cand[HID:], W_self_cand[HID:], bc)
    a0, a1 = _sc_cand(tc, src2, dst2, zrow)
    new = _stage_c(a0, a1, sc, u, st, dp)
    return new[:N]
